# baseline (speedup 1.0000x reference)
import functools

import jax
import jax.numpy as jnp
from jax import lax
from jax.experimental import pallas as pl
from jax.experimental.pallas import tpu as pltpu
from jax.experimental.pallas import tpu_sc as plsc

N = 10000
E = 160000
DB = 2048
DBH = 1024
DF = 512
H = 32

NW = 32
EPW = 5024
E_PAD = NW * EPW
NCH = EPW // 16
C = 16


def _mlp_q_body(x_ref, W1_ref, b1_ref, g_ref, be_ref, a_ref, W2_ref, b2_ref,
                Wb_ref, bb_ref, Pb_ref, Wf_ref, bf_ref, Pf_ref,
                qb_ref, qf_ref):
    h = jnp.dot(x_ref[...], W1_ref[...], preferred_element_type=jnp.float32)
    h = h + b1_ref[...]
    mu = jnp.mean(h, axis=0)
    var = jnp.mean((h - mu) ** 2, axis=0)
    h = (h - mu) / jnp.sqrt(var + 1e-5) * g_ref[...] + be_ref[...]
    h = jnp.where(h >= 0, h, a_ref[0, 0] * h)
    h = jnp.dot(h, W2_ref[...], preferred_element_type=jnp.float32) + b2_ref[...]
    wb = jnp.dot(Wb_ref[...], Pb_ref[...], preferred_element_type=jnp.float32)
    wf = jnp.dot(Wf_ref[...], Pf_ref[...], preferred_element_type=jnp.float32)
    qb_ref[...] = (jnp.dot(h, wb, preferred_element_type=jnp.float32)
                   + jnp.dot(bb_ref[...], Pb_ref[...], preferred_element_type=jnp.float32))
    qf_ref[...] = (jnp.dot(h, wf, preferred_element_type=jnp.float32)
                   + jnp.dot(bf_ref[...], Pf_ref[...], preferred_element_type=jnp.float32))


def _norm_body_body(v_ref, q_ref, vq_lo_ref, vq_hi_ref, vn_lo_ref, vn_hi_ref):
    v = v_ref[...]
    ss = jnp.sum(v * v, axis=1, keepdims=True)
    inv = 1.0 / (jnp.sqrt(ss) + 1e-8)
    vn = v * inv
    vq = vn * q_ref[...]
    vq_lo_ref[...] = vq[:, :DBH]
    vq_hi_ref[...] = vq[:, DBH:]
    vn_lo_ref[...] = vn[:, :DBH]
    vn_hi_ref[...] = vn[:, DBH:]


def _norm_face_body(v_ref, q_ref, vq_ref, vn_ref):
    v = v_ref[...]
    ss = jnp.sum(v * v, axis=1, keepdims=True)
    inv = 1.0 / (jnp.sqrt(ss) + 1e-8)
    vn = v * inv
    vq_ref[...] = vn * q_ref[...]
    vn_ref[...] = vn


def _fin_body(p_ref, pb_ref, pf_ref, o_ref):
    o_ref[...] = jnp.sum(p_ref[...], axis=0) + pb_ref[0] + pf_ref[0]


def _dots16(s_ref, d_ref, dh):
    lanes = lax.iota(jnp.int32, 16)
    U = 8
    def jstep(jb, accs):
        a0, a1 = accs
        jv = jnp.full((16,), jb * U, jnp.int32)
        for u in range(U):
            idx = jv + u
            sv = plsc.load_gather(s_ref, [lanes, idx])
            dv = plsc.load_gather(d_ref, [lanes, idx])
            if u % 2 == 0:
                a0 = a0 + sv * dv
            else:
                a1 = a1 + sv * dv
        return (a0, a1)
    z = jnp.zeros((16,), jnp.float32)
    a0, a1 = lax.fori_loop(0, dh // U, jstep, (z, z))
    return a0 + a1


def _sc_edge_kernel(vqb_lo, vqb_hi, vnb_lo, vnb_hi, vqf, vnf,
                    srcb, dstb, srcf, dstf, out_hbm,
                    sb0, sb1, db0, db1, sf0, sf1, df0, df1,
                    sib, dib, sif, dif, acc_v,
                    semb0, semb1, semf0, semf1):
    wid = lax.axis_index("s") * 2 + lax.axis_index("c")
    ebase = wid * EPW
    lanes = lax.iota(jnp.int32, 16)
    zero16 = jnp.zeros((16,), jnp.float32)

    def zstep(i, _):
        acc_v[pl.ds(i * 16, 16)] = zero16
        return 0
    lax.fori_loop(0, N // 16, zstep, 0)

    pltpu.sync_copy(srcb.at[pl.ds(ebase, EPW)], sib)
    pltpu.sync_copy(dstb.at[pl.ds(ebase, EPW)], dib)
    pltpu.sync_copy(srcf.at[pl.ds(ebase, EPW)], sif)
    pltpu.sync_copy(dstf.at[pl.ds(ebase, EPW)], dif)

    sbufs = (sb0, sb1)
    dbufs = (db0, db1)
    bsems = (semb0, semb1)
    vq_tabs = (vqb_lo, vqb_hi)
    vn_tabs = (vnb_lo, vnb_hi)

    def issue_b(c, h):
        si = sib.at[pl.ds(c * 16, 16)]
        di = dib.at[pl.ds(c * 16, 16)]
        pltpu.async_copy(vq_tabs[h].at[si], sbufs[h], bsems[h])
        pltpu.async_copy(vn_tabs[h].at[di], dbufs[h], bsems[h])

    def wait_b(c, h):
        si = sib.at[pl.ds(c * 16, 16)]
        di = dib.at[pl.ds(c * 16, 16)]
        pltpu.make_async_copy(vq_tabs[h].at[si], sbufs[h], bsems[h]).wait()
        pltpu.make_async_copy(vn_tabs[h].at[di], dbufs[h], bsems[h]).wait()

    issue_b(0, 0)
    issue_b(0, 1)

    def bstep(c, _):
        dots = zero16
        for h in range(2):
            wait_b(c, h)
            dots = _dots16(sbufs[h], dbufs[h], DBH) + dots

            @pl.when(c + 1 < NCH)
            def _():
                issue_b(c + 1, h)
        didx = dib[pl.ds(c * 16, 16)]
        mask = (ebase + c * 16 + lanes) < E
        plsc.addupdate_scatter(acc_v, [didx], dots, mask=mask)
        return 0
    lax.fori_loop(0, NCH, bstep, 0)

    fsbufs = (sf0, sf1)
    fdbufs = (df0, df1)
    fsems = (semf0, semf1)

    def issue_f(c, h):
        si = sif.at[pl.ds(c * 16, 16)]
        di = dif.at[pl.ds(c * 16, 16)]
        pltpu.async_copy(vqf.at[si], fsbufs[h], fsems[h])
        pltpu.async_copy(vnf.at[di], fdbufs[h], fsems[h])

    def wait_f(c, h):
        si = sif.at[pl.ds(c * 16, 16)]
        di = dif.at[pl.ds(c * 16, 16)]
        pltpu.make_async_copy(vqf.at[si], fsbufs[h], fsems[h]).wait()
        pltpu.make_async_copy(vnf.at[di], fdbufs[h], fsems[h]).wait()

    issue_f(0, 0)
    issue_f(1, 1)

    def fstep(cc, _):
        for h in range(2):
            c = cc * 2 + h
            wait_f(c, h)
            dots = _dots16(fsbufs[h], fdbufs[h], DF)

            @pl.when(c + 2 < NCH)
            def _():
                issue_f(c + 2, h)
            didx = dif[pl.ds(c * 16, 16)]
            mask = (ebase + c * 16 + lanes) < E
            plsc.addupdate_scatter(acc_v, [didx], dots, mask=mask)
        return 0
    lax.fori_loop(0, NCH // 2, fstep, 0)

    pltpu.sync_copy(acc_v, out_hbm.at[wid])


def kernel(x, edge_index_body, edge_index_face, visual_body, visual_face,
           W1, b1, bn_gamma, bn_beta, prelu_a, W2, b2,
           Wb, bb, Wf, bf, Pb, pb, Pf, pf):
    f32 = jnp.float32

    qb, qf = pl.pallas_call(
        _mlp_q_body,
        out_shape=(jax.ShapeDtypeStruct((N, 1), f32),
                   jax.ShapeDtypeStruct((N, 1), f32)),
    )(x, W1, b1, bn_gamma, bn_beta, prelu_a.reshape(1, 1).astype(f32),
      W2, b2, Wb, bb, Pb, Wf, bf, Pf)

    RB = 400
    vqb_lo, vqb_hi, vnb_lo, vnb_hi = pl.pallas_call(
        _norm_body_body,
        grid=(N // RB,),
        in_specs=[pl.BlockSpec((RB, DB), lambda i: (i, 0)),
                  pl.BlockSpec((RB, 1), lambda i: (i, 0))],
        out_specs=[pl.BlockSpec((RB, DBH), lambda i: (i, 0))] * 4,
        out_shape=(jax.ShapeDtypeStruct((N, DBH), f32),) * 4,
    )(visual_body, qb)

    RF = 1000
    vqf, vnf = pl.pallas_call(
        _norm_face_body,
        grid=(N // RF,),
        in_specs=[pl.BlockSpec((RF, DF), lambda i: (i, 0)),
                  pl.BlockSpec((RF, 1), lambda i: (i, 0))],
        out_specs=[pl.BlockSpec((RF, DF), lambda i: (i, 0))] * 2,
        out_shape=(jax.ShapeDtypeStruct((N, DF), f32),) * 2,
    )(visual_face, qf)

    pad = E_PAD - E
    zpad = jnp.zeros((pad,), jnp.int32)
    srcb = jnp.concatenate([edge_index_body[0], zpad])
    dstb = jnp.concatenate([edge_index_body[1], zpad])
    srcf = jnp.concatenate([edge_index_face[0], zpad])
    dstf = jnp.concatenate([edge_index_face[1], zpad])

    mesh = plsc.VectorSubcoreMesh(core_axis_name="c", subcore_axis_name="s")
    partial = pl.kernel(
        _sc_edge_kernel,
        out_type=jax.ShapeDtypeStruct((NW, N), f32),
        mesh=mesh,
        compiler_params=pltpu.CompilerParams(use_tc_tiling_on_sc=False, needs_layout_passes=False),
        scratch_types=[
            pltpu.VMEM((C, DBH), f32), pltpu.VMEM((C, DBH), f32),
            pltpu.VMEM((C, DBH), f32), pltpu.VMEM((C, DBH), f32),
            pltpu.VMEM((C, DF), f32), pltpu.VMEM((C, DF), f32),
            pltpu.VMEM((C, DF), f32), pltpu.VMEM((C, DF), f32),
            pltpu.VMEM((EPW,), jnp.int32), pltpu.VMEM((EPW,), jnp.int32),
            pltpu.VMEM((EPW,), jnp.int32), pltpu.VMEM((EPW,), jnp.int32),
            pltpu.VMEM((N,), f32),
            pltpu.SemaphoreType.DMA, pltpu.SemaphoreType.DMA,
            pltpu.SemaphoreType.DMA, pltpu.SemaphoreType.DMA,
        ],
    )(vqb_lo, vqb_hi, vnb_lo, vnb_hi, vqf, vnf, srcb, dstb, srcf, dstf)

    out = pl.pallas_call(
        _fin_body,
        out_shape=jax.ShapeDtypeStruct((N,), f32),
    )(partial, pb, pf)
    return out


# speedup vs baseline: 6.9253x; 6.9253x over previous
import functools

import jax
import jax.numpy as jnp
from jax import lax
from jax.experimental import pallas as pl
from jax.experimental.pallas import tpu as pltpu
from jax.experimental.pallas import tpu_sc as plsc

N = 10000
E = 160000
DB = 2048
DBH = 1024
DF = 512
H = 32

NW = 32
EPW = 5024
E_PAD = NW * EPW
NCH = EPW // 16
C = 16


def _mlp_q_body(x_ref, W1_ref, b1_ref, g_ref, be_ref, a_ref, W2_ref, b2_ref,
                Wb_ref, bb_ref, Pb_ref, Wf_ref, bf_ref, Pf_ref,
                qb_ref, qf_ref):
    h = jnp.dot(x_ref[...], W1_ref[...], preferred_element_type=jnp.float32)
    h = h + b1_ref[...]
    mu = jnp.mean(h, axis=0)
    var = jnp.mean((h - mu) ** 2, axis=0)
    h = (h - mu) / jnp.sqrt(var + 1e-5) * g_ref[...] + be_ref[...]
    h = jnp.where(h >= 0, h, a_ref[0, 0] * h)
    h = jnp.dot(h, W2_ref[...], preferred_element_type=jnp.float32) + b2_ref[...]
    wb = jnp.dot(Wb_ref[...], Pb_ref[...], preferred_element_type=jnp.float32)
    wf = jnp.dot(Wf_ref[...], Pf_ref[...], preferred_element_type=jnp.float32)
    qb_ref[...] = (jnp.dot(h, wb, preferred_element_type=jnp.float32)
                   + jnp.dot(bb_ref[...], Pb_ref[...], preferred_element_type=jnp.float32))
    qf_ref[...] = (jnp.dot(h, wf, preferred_element_type=jnp.float32)
                   + jnp.dot(bf_ref[...], Pf_ref[...], preferred_element_type=jnp.float32))


def _norm_body_body(v_ref, q_ref, vq_lo_ref, vq_hi_ref, vn_lo_ref, vn_hi_ref):
    v = v_ref[...]
    ss = jnp.sum(v * v, axis=1, keepdims=True)
    inv = 1.0 / (jnp.sqrt(ss) + 1e-8)
    vn = v * inv
    vq = vn * q_ref[...]
    vq_lo_ref[...] = vq[:, :DBH]
    vq_hi_ref[...] = vq[:, DBH:]
    vn_lo_ref[...] = vn[:, :DBH]
    vn_hi_ref[...] = vn[:, DBH:]


def _norm_face_body(v_ref, q_ref, vq_ref, vn_ref):
    v = v_ref[...]
    ss = jnp.sum(v * v, axis=1, keepdims=True)
    inv = 1.0 / (jnp.sqrt(ss) + 1e-8)
    vn = v * inv
    vq_ref[...] = vn * q_ref[...]
    vn_ref[...] = vn


def _fin_body(p_ref, pb_ref, pf_ref, o_ref):
    o_ref[...] = jnp.sum(p_ref[...], axis=0) + pb_ref[0] + pf_ref[0]


def _allsum16(t, lanes):
    dnums = lax.GatherDimensionNumbers(
        offset_dims=(), collapsed_slice_dims=(0,), start_index_map=(0,))
    for k in (8, 4, 2, 1):
        perm = lax.gather(t, (lanes ^ k)[:, None], dimension_numbers=dnums,
                          slice_sizes=(1,),
                          mode=lax.GatherScatterMode.PROMISE_IN_BOUNDS)
        t = t + perm
    return t


def _dots16(s_ref, d_ref, dh):
    lanes = lax.iota(jnp.int32, 16)
    z = jnp.zeros((16,), jnp.float32)
    U = 8

    def estep(i, res):
        def jstep(jb, accs):
            a0, a1 = accs
            for u in range(U):
                o = (jb * U + u) * 16
                sv = s_ref[i, pl.ds(o, 16)]
                dv = d_ref[i, pl.ds(o, 16)]
                if u % 2 == 0:
                    a0 = a0 + sv * dv
                else:
                    a1 = a1 + sv * dv
            return (a0, a1)
        a0, a1 = lax.fori_loop(0, dh // (16 * U), jstep, (z, z))
        t = _allsum16(a0 + a1, lanes)
        return jnp.where(lanes == i, t, res)

    return lax.fori_loop(0, C, estep, z)


def _sc_edge_kernel(vqb_lo, vqb_hi, vnb_lo, vnb_hi, vqf, vnf,
                    srcb, dstb, srcf, dstf, out_hbm,
                    sb0, sb1, db0, db1, sf0, sf1, df0, df1,
                    sib, dib, acc_v,
                    semb0, semb1, semf0, semf1):
    wid = lax.axis_index("s") * 2 + lax.axis_index("c")
    ebase = wid * EPW
    lanes = lax.iota(jnp.int32, 16)
    zero16 = jnp.zeros((16,), jnp.float32)

    def zstep(i, _):
        acc_v[pl.ds(i * 16, 16)] = zero16
        return 0
    lax.fori_loop(0, N // 16, zstep, 0)

    pltpu.sync_copy(srcb.at[pl.ds(ebase, EPW)], sib)
    pltpu.sync_copy(dstb.at[pl.ds(ebase, EPW)], dib)

    sbufs = (sb0, sb1)
    dbufs = (db0, db1)
    bsems = (semb0, semb1)
    vq_tabs = (vqb_lo, vqb_hi)
    vn_tabs = (vnb_lo, vnb_hi)

    def issue_b(c, h):
        si = sib.at[pl.ds(c * 16, 16)]
        di = dib.at[pl.ds(c * 16, 16)]
        pltpu.async_copy(vq_tabs[h].at[si], sbufs[h], bsems[h])
        pltpu.async_copy(vn_tabs[h].at[di], dbufs[h], bsems[h])

    def wait_b(c, h):
        si = sib.at[pl.ds(c * 16, 16)]
        di = dib.at[pl.ds(c * 16, 16)]
        pltpu.make_async_copy(vq_tabs[h].at[si], sbufs[h], bsems[h]).wait()
        pltpu.make_async_copy(vn_tabs[h].at[di], dbufs[h], bsems[h]).wait()

    issue_b(0, 0)
    issue_b(0, 1)

    def bstep(c, _):
        dots = zero16
        for h in range(2):
            wait_b(c, h)
            dots = _dots16(sbufs[h], dbufs[h], DBH) + dots

            @pl.when(c + 1 < NCH)
            def _():
                issue_b(c + 1, h)
        didx = dib[pl.ds(c * 16, 16)]
        mask = (ebase + c * 16 + lanes) < E
        plsc.addupdate_scatter(acc_v, [didx], dots, mask=mask)
        return 0
    lax.fori_loop(0, NCH, bstep, 0)

    fsbufs = (sf0, sf1)
    fdbufs = (df0, df1)
    fsems = (semf0, semf1)
    pltpu.sync_copy(srcf.at[pl.ds(ebase, EPW)], sib)
    pltpu.sync_copy(dstf.at[pl.ds(ebase, EPW)], dib)

    def issue_f(c, h):
        si = sib.at[pl.ds(c * 16, 16)]
        di = dib.at[pl.ds(c * 16, 16)]
        pltpu.async_copy(vqf.at[si], fsbufs[h], fsems[h])
        pltpu.async_copy(vnf.at[di], fdbufs[h], fsems[h])

    def wait_f(c, h):
        si = sib.at[pl.ds(c * 16, 16)]
        di = dib.at[pl.ds(c * 16, 16)]
        pltpu.make_async_copy(vqf.at[si], fsbufs[h], fsems[h]).wait()
        pltpu.make_async_copy(vnf.at[di], fdbufs[h], fsems[h]).wait()

    issue_f(0, 0)
    issue_f(1, 1)

    def fstep(cc, _):
        for h in range(2):
            c = cc * 2 + h
            wait_f(c, h)
            dots = _dots16(fsbufs[h], fdbufs[h], DF)

            @pl.when(c + 2 < NCH)
            def _():
                issue_f(c + 2, h)
            didx = dib[pl.ds(c * 16, 16)]
            mask = (ebase + c * 16 + lanes) < E
            plsc.addupdate_scatter(acc_v, [didx], dots, mask=mask)
        return 0
    lax.fori_loop(0, NCH // 2, fstep, 0)

    pltpu.sync_copy(acc_v, out_hbm.at[wid])


def kernel(x, edge_index_body, edge_index_face, visual_body, visual_face,
           W1, b1, bn_gamma, bn_beta, prelu_a, W2, b2,
           Wb, bb, Wf, bf, Pb, pb, Pf, pf):
    f32 = jnp.float32

    qb, qf = pl.pallas_call(
        _mlp_q_body,
        out_shape=(jax.ShapeDtypeStruct((N, 1), f32),
                   jax.ShapeDtypeStruct((N, 1), f32)),
    )(x, W1, b1, bn_gamma, bn_beta, prelu_a.reshape(1, 1).astype(f32),
      W2, b2, Wb, bb, Pb, Wf, bf, Pf)

    RB = 400
    vqb_lo, vqb_hi, vnb_lo, vnb_hi = pl.pallas_call(
        _norm_body_body,
        grid=(N // RB,),
        in_specs=[pl.BlockSpec((RB, DB), lambda i: (i, 0)),
                  pl.BlockSpec((RB, 1), lambda i: (i, 0))],
        out_specs=[pl.BlockSpec((RB, DBH), lambda i: (i, 0))] * 4,
        out_shape=(jax.ShapeDtypeStruct((N, DBH), f32),) * 4,
    )(visual_body, qb)

    RF = 1000
    vqf, vnf = pl.pallas_call(
        _norm_face_body,
        grid=(N // RF,),
        in_specs=[pl.BlockSpec((RF, DF), lambda i: (i, 0)),
                  pl.BlockSpec((RF, 1), lambda i: (i, 0))],
        out_specs=[pl.BlockSpec((RF, DF), lambda i: (i, 0))] * 2,
        out_shape=(jax.ShapeDtypeStruct((N, DF), f32),) * 2,
    )(visual_face, qf)

    pad = E_PAD - E
    zpad = jnp.zeros((pad,), jnp.int32)
    srcb = jnp.concatenate([edge_index_body[0], zpad])
    dstb = jnp.concatenate([edge_index_body[1], zpad])
    srcf = jnp.concatenate([edge_index_face[0], zpad])
    dstf = jnp.concatenate([edge_index_face[1], zpad])

    mesh = plsc.VectorSubcoreMesh(core_axis_name="c", subcore_axis_name="s")
    partial = pl.kernel(
        _sc_edge_kernel,
        out_type=jax.ShapeDtypeStruct((NW, N), f32),
        mesh=mesh,
        compiler_params=pltpu.CompilerParams(use_tc_tiling_on_sc=False, needs_layout_passes=False),
        scratch_types=[
            pltpu.VMEM((C, DBH), f32), pltpu.VMEM((C, DBH), f32),
            pltpu.VMEM((C, DBH), f32), pltpu.VMEM((C, DBH), f32),
            pltpu.VMEM((C, DF), f32), pltpu.VMEM((C, DF), f32),
            pltpu.VMEM((C, DF), f32), pltpu.VMEM((C, DF), f32),
            pltpu.VMEM((EPW,), jnp.int32), pltpu.VMEM((EPW,), jnp.int32),
            pltpu.VMEM((N,), f32),
            pltpu.SemaphoreType.DMA, pltpu.SemaphoreType.DMA,
            pltpu.SemaphoreType.DMA, pltpu.SemaphoreType.DMA,
        ],
    )(vqb_lo, vqb_hi, vnb_lo, vnb_hi, vqf, vnf, srcb, dstb, srcf, dstf)

    out = pl.pallas_call(
        _fin_body,
        out_shape=jax.ShapeDtypeStruct((N,), f32),
    )(partial, pb, pf)
    return out


# speedup vs baseline: 7.8955x; 1.1401x over previous
import functools

import jax
import jax.numpy as jnp
from jax import lax
from jax.experimental import pallas as pl
from jax.experimental.pallas import tpu as pltpu
from jax.experimental.pallas import tpu_sc as plsc

N = 10000
E = 160000
DB = 2048
DBH = 1024
DF = 512
H = 32

NW = 32
EPW = 5024
E_PAD = NW * EPW
NCH = EPW // 16
C = 16


def _mlp_q_body(x_ref, W1_ref, b1_ref, g_ref, be_ref, a_ref, W2_ref, b2_ref,
                Wb_ref, bb_ref, Pb_ref, Wf_ref, bf_ref, Pf_ref,
                qb_ref, qf_ref):
    h = jnp.dot(x_ref[...], W1_ref[...], preferred_element_type=jnp.float32)
    h = h + b1_ref[...]
    mu = jnp.mean(h, axis=0)
    var = jnp.mean((h - mu) ** 2, axis=0)
    h = (h - mu) / jnp.sqrt(var + 1e-5) * g_ref[...] + be_ref[...]
    h = jnp.where(h >= 0, h, a_ref[0, 0] * h)
    h = jnp.dot(h, W2_ref[...], preferred_element_type=jnp.float32) + b2_ref[...]
    wb = jnp.dot(Wb_ref[...], Pb_ref[...], preferred_element_type=jnp.float32)
    wf = jnp.dot(Wf_ref[...], Pf_ref[...], preferred_element_type=jnp.float32)
    qb_ref[...] = (jnp.dot(h, wb, preferred_element_type=jnp.float32)
                   + jnp.dot(bb_ref[...], Pb_ref[...], preferred_element_type=jnp.float32))
    qf_ref[...] = (jnp.dot(h, wf, preferred_element_type=jnp.float32)
                   + jnp.dot(bf_ref[...], Pf_ref[...], preferred_element_type=jnp.float32))


def _norm_body_body(v_ref, q_ref, vq_lo_ref, vq_hi_ref, vn_lo_ref, vn_hi_ref):
    v = v_ref[...]
    ss = jnp.sum(v * v, axis=1, keepdims=True)
    inv = 1.0 / (jnp.sqrt(ss) + 1e-8)
    vn = v * inv
    vq = vn * q_ref[...]
    vq_lo_ref[...] = vq[:, :DBH].astype(jnp.bfloat16)
    vq_hi_ref[...] = vq[:, DBH:].astype(jnp.bfloat16)
    vn_lo_ref[...] = vn[:, :DBH].astype(jnp.bfloat16)
    vn_hi_ref[...] = vn[:, DBH:].astype(jnp.bfloat16)


def _norm_face_body(v_ref, q_ref, vq_ref, vn_ref):
    v = v_ref[...]
    ss = jnp.sum(v * v, axis=1, keepdims=True)
    inv = 1.0 / (jnp.sqrt(ss) + 1e-8)
    vn = v * inv
    vq_ref[...] = (vn * q_ref[...]).astype(jnp.bfloat16)
    vn_ref[...] = vn.astype(jnp.bfloat16)


def _fin_body(p_ref, pb_ref, pf_ref, o_ref):
    o_ref[...] = jnp.sum(p_ref[...], axis=0) + pb_ref[0] + pf_ref[0]


def _allsum16(t, lanes):
    dnums = lax.GatherDimensionNumbers(
        offset_dims=(), collapsed_slice_dims=(0,), start_index_map=(0,))
    for k in (8, 4, 2, 1):
        perm = lax.gather(t, (lanes ^ k)[:, None], dimension_numbers=dnums,
                          slice_sizes=(1,),
                          mode=lax.GatherScatterMode.PROMISE_IN_BOUNDS)
        t = t + perm
    return t


def _dots16(s_ref, d_ref, dh):
    lanes = lax.iota(jnp.int32, 16)
    z = jnp.zeros((16,), jnp.float32)
    U = 8

    def estep(i, res):
        def jstep(jb, accs):
            a0, a1 = accs
            for u in range(U):
                o = (jb * U + u) * 32
                sv = s_ref[i, pl.ds(o, 32)]
                dv = d_ref[i, pl.ds(o, 32)]
                s0, s1 = plsc.unpack(sv, format=plsc.PackFormat.INTERLEAVED)
                d0, d1 = plsc.unpack(dv, format=plsc.PackFormat.INTERLEAVED)
                a0 = a0 + s0 * d0
                a1 = a1 + s1 * d1
            return (a0, a1)
        a0, a1 = lax.fori_loop(0, dh // (32 * U), jstep, (z, z))
        t = _allsum16(a0 + a1, lanes)
        return jnp.where(lanes == i, t, res)

    return lax.fori_loop(0, C, estep, z)


def _sc_edge_kernel(vqb_lo, vqb_hi, vnb_lo, vnb_hi, vqf, vnf,
                    srcb, dstb, srcf, dstf, out_hbm,
                    sb0, sb1, db0, db1, sf0, sf1, df0, df1,
                    sib, dib, acc_v,
                    semb0, semb1, semf0, semf1):
    wid = lax.axis_index("s") * 2 + lax.axis_index("c")
    ebase = wid * EPW
    lanes = lax.iota(jnp.int32, 16)
    zero16 = jnp.zeros((16,), jnp.float32)

    def zstep(i, _):
        acc_v[pl.ds(i * 16, 16)] = zero16
        return 0
    lax.fori_loop(0, N // 16, zstep, 0)

    pltpu.sync_copy(srcb.at[pl.ds(ebase, EPW)], sib)
    pltpu.sync_copy(dstb.at[pl.ds(ebase, EPW)], dib)

    sbufs = (sb0, sb1)
    dbufs = (db0, db1)
    bsems = (semb0, semb1)
    vq_tabs = (vqb_lo, vqb_hi)
    vn_tabs = (vnb_lo, vnb_hi)

    def issue_b(c, h):
        si = sib.at[pl.ds(c * 16, 16)]
        di = dib.at[pl.ds(c * 16, 16)]
        pltpu.async_copy(vq_tabs[h].at[si], sbufs[h], bsems[h])
        pltpu.async_copy(vn_tabs[h].at[di], dbufs[h], bsems[h])

    def wait_b(c, h):
        si = sib.at[pl.ds(c * 16, 16)]
        di = dib.at[pl.ds(c * 16, 16)]
        pltpu.make_async_copy(vq_tabs[h].at[si], sbufs[h], bsems[h]).wait()
        pltpu.make_async_copy(vn_tabs[h].at[di], dbufs[h], bsems[h]).wait()

    issue_b(0, 0)
    issue_b(0, 1)

    def bstep(c, _):
        dots = zero16
        for h in range(2):
            wait_b(c, h)
            dots = _dots16(sbufs[h], dbufs[h], DBH) + dots

            @pl.when(c + 1 < NCH)
            def _():
                issue_b(c + 1, h)
        didx = dib[pl.ds(c * 16, 16)]
        mask = (ebase + c * 16 + lanes) < E
        plsc.addupdate_scatter(acc_v, [didx], dots, mask=mask)
        return 0
    lax.fori_loop(0, NCH, bstep, 0)

    fsbufs = (sf0, sf1)
    fdbufs = (df0, df1)
    fsems = (semf0, semf1)
    pltpu.sync_copy(srcf.at[pl.ds(ebase, EPW)], sib)
    pltpu.sync_copy(dstf.at[pl.ds(ebase, EPW)], dib)

    def issue_f(c, h):
        si = sib.at[pl.ds(c * 16, 16)]
        di = dib.at[pl.ds(c * 16, 16)]
        pltpu.async_copy(vqf.at[si], fsbufs[h], fsems[h])
        pltpu.async_copy(vnf.at[di], fdbufs[h], fsems[h])

    def wait_f(c, h):
        si = sib.at[pl.ds(c * 16, 16)]
        di = dib.at[pl.ds(c * 16, 16)]
        pltpu.make_async_copy(vqf.at[si], fsbufs[h], fsems[h]).wait()
        pltpu.make_async_copy(vnf.at[di], fdbufs[h], fsems[h]).wait()

    issue_f(0, 0)
    issue_f(1, 1)

    def fstep(cc, _):
        for h in range(2):
            c = cc * 2 + h
            wait_f(c, h)
            dots = _dots16(fsbufs[h], fdbufs[h], DF)

            @pl.when(c + 2 < NCH)
            def _():
                issue_f(c + 2, h)
            didx = dib[pl.ds(c * 16, 16)]
            mask = (ebase + c * 16 + lanes) < E
            plsc.addupdate_scatter(acc_v, [didx], dots, mask=mask)
        return 0
    lax.fori_loop(0, NCH // 2, fstep, 0)

    pltpu.sync_copy(acc_v, out_hbm.at[wid])


def kernel(x, edge_index_body, edge_index_face, visual_body, visual_face,
           W1, b1, bn_gamma, bn_beta, prelu_a, W2, b2,
           Wb, bb, Wf, bf, Pb, pb, Pf, pf):
    f32 = jnp.float32

    qb, qf = pl.pallas_call(
        _mlp_q_body,
        out_shape=(jax.ShapeDtypeStruct((N, 1), f32),
                   jax.ShapeDtypeStruct((N, 1), f32)),
    )(x, W1, b1, bn_gamma, bn_beta, prelu_a.reshape(1, 1).astype(f32),
      W2, b2, Wb, bb, Pb, Wf, bf, Pf)

    RB = 400
    vqb_lo, vqb_hi, vnb_lo, vnb_hi = pl.pallas_call(
        _norm_body_body,
        grid=(N // RB,),
        in_specs=[pl.BlockSpec((RB, DB), lambda i: (i, 0)),
                  pl.BlockSpec((RB, 1), lambda i: (i, 0))],
        out_specs=[pl.BlockSpec((RB, DBH), lambda i: (i, 0))] * 4,
        out_shape=(jax.ShapeDtypeStruct((N, DBH), jnp.bfloat16),) * 4,
    )(visual_body, qb)

    RF = 2000
    vqf, vnf = pl.pallas_call(
        _norm_face_body,
        grid=(N // RF,),
        in_specs=[pl.BlockSpec((RF, DF), lambda i: (i, 0)),
                  pl.BlockSpec((RF, 1), lambda i: (i, 0))],
        out_specs=[pl.BlockSpec((RF, DF), lambda i: (i, 0))] * 2,
        out_shape=(jax.ShapeDtypeStruct((N, DF), jnp.bfloat16),) * 2,
    )(visual_face, qf)

    pad = E_PAD - E
    zpad = jnp.zeros((pad,), jnp.int32)
    srcb = jnp.concatenate([edge_index_body[0], zpad])
    dstb = jnp.concatenate([edge_index_body[1], zpad])
    srcf = jnp.concatenate([edge_index_face[0], zpad])
    dstf = jnp.concatenate([edge_index_face[1], zpad])

    mesh = plsc.VectorSubcoreMesh(core_axis_name="c", subcore_axis_name="s")
    partial = pl.kernel(
        _sc_edge_kernel,
        out_type=jax.ShapeDtypeStruct((NW, N), f32),
        mesh=mesh,
        compiler_params=pltpu.CompilerParams(use_tc_tiling_on_sc=False, needs_layout_passes=False),
        scratch_types=[
            pltpu.VMEM((C, DBH), jnp.bfloat16), pltpu.VMEM((C, DBH), jnp.bfloat16),
            pltpu.VMEM((C, DBH), jnp.bfloat16), pltpu.VMEM((C, DBH), jnp.bfloat16),
            pltpu.VMEM((C, DF), jnp.bfloat16), pltpu.VMEM((C, DF), jnp.bfloat16),
            pltpu.VMEM((C, DF), jnp.bfloat16), pltpu.VMEM((C, DF), jnp.bfloat16),
            pltpu.VMEM((EPW,), jnp.int32), pltpu.VMEM((EPW,), jnp.int32),
            pltpu.VMEM((N,), f32),
            pltpu.SemaphoreType.DMA, pltpu.SemaphoreType.DMA,
            pltpu.SemaphoreType.DMA, pltpu.SemaphoreType.DMA,
        ],
    )(vqb_lo, vqb_hi, vnb_lo, vnb_hi, vqf, vnf, srcb, dstb, srcf, dstf)

    out = pl.pallas_call(
        _fin_body,
        out_shape=jax.ShapeDtypeStruct((N,), f32),
    )(partial, pb, pf)
    return out


# speedup vs baseline: 9.3386x; 1.1828x over previous
import functools

import jax
import jax.numpy as jnp
from jax import lax
from jax.experimental import pallas as pl
from jax.experimental.pallas import tpu as pltpu
from jax.experimental.pallas import tpu_sc as plsc

N = 10000
E = 160000
DB = 2048
DBH = 1024
DF = 512
H = 32

NW = 32
EPW = 5024
E_PAD = NW * EPW
NCH = EPW // 16
C = 16


def _mlp_q_body(x_ref, W1_ref, b1_ref, g_ref, be_ref, a_ref, W2_ref, b2_ref,
                Wb_ref, bb_ref, Pb_ref, Wf_ref, bf_ref, Pf_ref,
                qb_ref, qf_ref):
    h = jnp.dot(x_ref[...], W1_ref[...], preferred_element_type=jnp.float32)
    h = h + b1_ref[...]
    mu = jnp.mean(h, axis=0)
    var = jnp.mean((h - mu) ** 2, axis=0)
    h = (h - mu) / jnp.sqrt(var + 1e-5) * g_ref[...] + be_ref[...]
    h = jnp.where(h >= 0, h, a_ref[0, 0] * h)
    h = jnp.dot(h, W2_ref[...], preferred_element_type=jnp.float32) + b2_ref[...]
    wb = jnp.dot(Wb_ref[...], Pb_ref[...], preferred_element_type=jnp.float32)
    wf = jnp.dot(Wf_ref[...], Pf_ref[...], preferred_element_type=jnp.float32)
    qb_ref[...] = (jnp.dot(h, wb, preferred_element_type=jnp.float32)
                   + jnp.dot(bb_ref[...], Pb_ref[...], preferred_element_type=jnp.float32))
    qf_ref[...] = (jnp.dot(h, wf, preferred_element_type=jnp.float32)
                   + jnp.dot(bf_ref[...], Pf_ref[...], preferred_element_type=jnp.float32))


def _norm_body_body(v_ref, q_ref, vq_lo_ref, vq_hi_ref, vn_lo_ref, vn_hi_ref):
    v = v_ref[...]
    ss = jnp.sum(v * v, axis=1, keepdims=True)
    inv = 1.0 / (jnp.sqrt(ss) + 1e-8)
    vn = v * inv
    vq = vn * q_ref[...]
    vq_lo_ref[...] = vq[:, :DBH].astype(jnp.bfloat16)
    vq_hi_ref[...] = vq[:, DBH:].astype(jnp.bfloat16)
    vn_lo_ref[...] = vn[:, :DBH].astype(jnp.bfloat16)
    vn_hi_ref[...] = vn[:, DBH:].astype(jnp.bfloat16)


def _norm_face_body(v_ref, q_ref, vq_ref, vn_ref):
    v = v_ref[...]
    ss = jnp.sum(v * v, axis=1, keepdims=True)
    inv = 1.0 / (jnp.sqrt(ss) + 1e-8)
    vn = v * inv
    vq_ref[...] = (vn * q_ref[...]).astype(jnp.bfloat16)
    vn_ref[...] = vn.astype(jnp.bfloat16)


def _fin_body(p_ref, pb_ref, pf_ref, o_ref):
    o_ref[...] = jnp.sum(p_ref[...], axis=0) + pb_ref[0] + pf_ref[0]


def _allsum16(t, lanes):
    dnums = lax.GatherDimensionNumbers(
        offset_dims=(), collapsed_slice_dims=(0,), start_index_map=(0,))
    for k in (8, 4, 2, 1):
        perm = lax.gather(t, (lanes ^ k)[:, None], dimension_numbers=dnums,
                          slice_sizes=(1,),
                          mode=lax.GatherScatterMode.PROMISE_IN_BOUNDS)
        t = t + perm
    return t


def _dots16(s_ref, d_ref, dh):
    lanes = lax.iota(jnp.int32, 16)
    zf = jnp.zeros((16,), jnp.float32)
    zb = jnp.zeros((32,), jnp.bfloat16)
    nld = dh // 32

    def estep(i, res):
        f0, f1 = zf, zf
        a, b = zb, zb
        for u in range(nld):
            sv = s_ref[i, pl.ds(u * 32, 32)]
            dv = d_ref[i, pl.ds(u * 32, 32)]
            p = sv * dv
            if u % 2 == 0:
                a = a + p
            else:
                b = b + p
            if u % 8 == 7 or u == nld - 1:
                for acc in (a, b):
                    p0, p1 = plsc.unpack(acc, format=plsc.PackFormat.INTERLEAVED)
                    f0 = f0 + p0
                    f1 = f1 + p1
                a, b = zb, zb
        t = _allsum16(f0 + f1, lanes)
        return jnp.where(lanes == i, t, res)

    return lax.fori_loop(0, C, estep, zf)


def _sc_edge_kernel(vqb_lo, vqb_hi, vnb_lo, vnb_hi, vqf, vnf,
                    srcb, dstb, srcf, dstf, out_hbm,
                    sb0, sb1, db0, db1, sf0, sf1, df0, df1,
                    sib, dib, acc_v,
                    semb0, semb1, semf0, semf1):
    wid = lax.axis_index("s") * 2 + lax.axis_index("c")
    ebase = wid * EPW
    lanes = lax.iota(jnp.int32, 16)
    zero16 = jnp.zeros((16,), jnp.float32)

    def zstep(i, _):
        acc_v[pl.ds(i * 16, 16)] = zero16
        return 0
    lax.fori_loop(0, N // 16, zstep, 0)

    pltpu.sync_copy(srcb.at[pl.ds(ebase, EPW)], sib)
    pltpu.sync_copy(dstb.at[pl.ds(ebase, EPW)], dib)

    sbufs = (sb0, sb1)
    dbufs = (db0, db1)
    bsems = (semb0, semb1)
    vq_tabs = (vqb_lo, vqb_hi)
    vn_tabs = (vnb_lo, vnb_hi)

    def issue_b(c, h):
        si = sib.at[pl.ds(c * 16, 16)]
        di = dib.at[pl.ds(c * 16, 16)]
        pltpu.async_copy(vq_tabs[h].at[si], sbufs[h], bsems[h])
        pltpu.async_copy(vn_tabs[h].at[di], dbufs[h], bsems[h])

    def wait_b(c, h):
        si = sib.at[pl.ds(c * 16, 16)]
        di = dib.at[pl.ds(c * 16, 16)]
        pltpu.make_async_copy(vq_tabs[h].at[si], sbufs[h], bsems[h]).wait()
        pltpu.make_async_copy(vn_tabs[h].at[di], dbufs[h], bsems[h]).wait()

    issue_b(0, 0)
    issue_b(0, 1)

    def bstep(c, _):
        dots = zero16
        for h in range(2):
            wait_b(c, h)
            dots = _dots16(sbufs[h], dbufs[h], DBH) + dots

            @pl.when(c + 1 < NCH)
            def _():
                issue_b(c + 1, h)
        didx = dib[pl.ds(c * 16, 16)]
        mask = (ebase + c * 16 + lanes) < E
        plsc.addupdate_scatter(acc_v, [didx], dots, mask=mask)
        return 0
    lax.fori_loop(0, NCH, bstep, 0)

    fsbufs = (sf0, sf1)
    fdbufs = (df0, df1)
    fsems = (semf0, semf1)
    pltpu.sync_copy(srcf.at[pl.ds(ebase, EPW)], sib)
    pltpu.sync_copy(dstf.at[pl.ds(ebase, EPW)], dib)

    def issue_f(c, h):
        si = sib.at[pl.ds(c * 16, 16)]
        di = dib.at[pl.ds(c * 16, 16)]
        pltpu.async_copy(vqf.at[si], fsbufs[h], fsems[h])
        pltpu.async_copy(vnf.at[di], fdbufs[h], fsems[h])

    def wait_f(c, h):
        si = sib.at[pl.ds(c * 16, 16)]
        di = dib.at[pl.ds(c * 16, 16)]
        pltpu.make_async_copy(vqf.at[si], fsbufs[h], fsems[h]).wait()
        pltpu.make_async_copy(vnf.at[di], fdbufs[h], fsems[h]).wait()

    issue_f(0, 0)
    issue_f(1, 1)

    def fstep(cc, _):
        for h in range(2):
            c = cc * 2 + h
            wait_f(c, h)
            dots = _dots16(fsbufs[h], fdbufs[h], DF)

            @pl.when(c + 2 < NCH)
            def _():
                issue_f(c + 2, h)
            didx = dib[pl.ds(c * 16, 16)]
            mask = (ebase + c * 16 + lanes) < E
            plsc.addupdate_scatter(acc_v, [didx], dots, mask=mask)
        return 0
    lax.fori_loop(0, NCH // 2, fstep, 0)

    pltpu.sync_copy(acc_v, out_hbm.at[wid])


def kernel(x, edge_index_body, edge_index_face, visual_body, visual_face,
           W1, b1, bn_gamma, bn_beta, prelu_a, W2, b2,
           Wb, bb, Wf, bf, Pb, pb, Pf, pf):
    f32 = jnp.float32

    qb, qf = pl.pallas_call(
        _mlp_q_body,
        out_shape=(jax.ShapeDtypeStruct((N, 1), f32),
                   jax.ShapeDtypeStruct((N, 1), f32)),
    )(x, W1, b1, bn_gamma, bn_beta, prelu_a.reshape(1, 1).astype(f32),
      W2, b2, Wb, bb, Pb, Wf, bf, Pf)

    RB = 400
    vqb_lo, vqb_hi, vnb_lo, vnb_hi = pl.pallas_call(
        _norm_body_body,
        grid=(N // RB,),
        in_specs=[pl.BlockSpec((RB, DB), lambda i: (i, 0)),
                  pl.BlockSpec((RB, 1), lambda i: (i, 0))],
        out_specs=[pl.BlockSpec((RB, DBH), lambda i: (i, 0))] * 4,
        out_shape=(jax.ShapeDtypeStruct((N, DBH), jnp.bfloat16),) * 4,
    )(visual_body, qb)

    RF = 2000
    vqf, vnf = pl.pallas_call(
        _norm_face_body,
        grid=(N // RF,),
        in_specs=[pl.BlockSpec((RF, DF), lambda i: (i, 0)),
                  pl.BlockSpec((RF, 1), lambda i: (i, 0))],
        out_specs=[pl.BlockSpec((RF, DF), lambda i: (i, 0))] * 2,
        out_shape=(jax.ShapeDtypeStruct((N, DF), jnp.bfloat16),) * 2,
    )(visual_face, qf)

    pad = E_PAD - E
    zpad = jnp.zeros((pad,), jnp.int32)
    srcb = jnp.concatenate([edge_index_body[0], zpad])
    dstb = jnp.concatenate([edge_index_body[1], zpad])
    srcf = jnp.concatenate([edge_index_face[0], zpad])
    dstf = jnp.concatenate([edge_index_face[1], zpad])

    mesh = plsc.VectorSubcoreMesh(core_axis_name="c", subcore_axis_name="s")
    partial = pl.kernel(
        _sc_edge_kernel,
        out_type=jax.ShapeDtypeStruct((NW, N), f32),
        mesh=mesh,
        compiler_params=pltpu.CompilerParams(use_tc_tiling_on_sc=False, needs_layout_passes=False),
        scratch_types=[
            pltpu.VMEM((C, DBH), jnp.bfloat16), pltpu.VMEM((C, DBH), jnp.bfloat16),
            pltpu.VMEM((C, DBH), jnp.bfloat16), pltpu.VMEM((C, DBH), jnp.bfloat16),
            pltpu.VMEM((C, DF), jnp.bfloat16), pltpu.VMEM((C, DF), jnp.bfloat16),
            pltpu.VMEM((C, DF), jnp.bfloat16), pltpu.VMEM((C, DF), jnp.bfloat16),
            pltpu.VMEM((EPW,), jnp.int32), pltpu.VMEM((EPW,), jnp.int32),
            pltpu.VMEM((N,), f32),
            pltpu.SemaphoreType.DMA, pltpu.SemaphoreType.DMA,
            pltpu.SemaphoreType.DMA, pltpu.SemaphoreType.DMA,
        ],
    )(vqb_lo, vqb_hi, vnb_lo, vnb_hi, vqf, vnf, srcb, dstb, srcf, dstf)

    out = pl.pallas_call(
        _fin_body,
        out_shape=jax.ShapeDtypeStruct((N,), f32),
    )(partial, pb, pf)
    return out
